# Initial kernel scaffold; baseline (speedup 1.0000x reference)
import jax
import jax.numpy as jnp
from jax.experimental import pallas as pl


def kernel(current_hidden_states, layer_input, memory_keys, memory_values, Wq, bq, Wk, bk, Wv, bv, Wg1, bg1, Wg2, bg2, ln_gamma, ln_beta):
    raise NotImplementedError("write your pallas kernel here")


# speedup vs baseline: 2.4166x; 2.4166x over previous
import jax
import jax.numpy as jnp
from jax.experimental import pallas as pl
from jax.experimental.pallas import tpu as pltpu


def _fused_kernel(li_ref, ch_ref, wq_ref, bq_ref, mkT_ref, mv_ref,
                  wg1a_ref, wg1b_ref, bg1_ref, wg2_ref, bg2_ref,
                  gamma_ref, beta_ref, out_ref, gate_ref):
    li = li_ref[...]
    ch = ch_ref[...]

    q = jnp.dot(li, wq_ref[...], preferred_element_type=jnp.float32)
    q = q + bq_ref[...]

    sim = jnp.dot(q, mkT_ref[...], preferred_element_type=jnp.float32)
    m = jnp.max(sim, axis=-1, keepdims=True)
    e = jnp.exp(sim - m)
    attn = e / jnp.sum(e, axis=-1, keepdims=True)

    retr = jnp.dot(attn, mv_ref[...], preferred_element_type=jnp.float32)

    h = jnp.dot(li, wg1a_ref[...], preferred_element_type=jnp.float32)
    h = h + jnp.dot(retr, wg1b_ref[...], preferred_element_type=jnp.float32)
    h = jnp.maximum(h + bg1_ref[...], 0.0)

    glogit = jnp.sum(h * wg2_ref[...], axis=-1, keepdims=True) + bg2_ref[...]
    gate = jax.nn.sigmoid(glogit)

    upd = ch + gate * (retr - ch)
    mean = jnp.mean(upd, axis=-1, keepdims=True)
    xc = upd - mean
    var = jnp.mean(xc * xc, axis=-1, keepdims=True)
    out = xc * jax.lax.rsqrt(var + 1e-5) * gamma_ref[...] + beta_ref[...]

    out_ref[...] = out
    gate_ref[...] = gate.reshape(1, 1, -1)


def kernel(current_hidden_states, layer_input, memory_keys, memory_values,
           Wq, bq, Wk, bk, Wv, bv, Wg1, bg1, Wg2, bg2, ln_gamma, ln_beta):
    B, S, D = current_hidden_states.shape
    M = memory_keys.shape[0]
    H = Wg1.shape[1]
    N = B * S
    T = 512
    grid = (N // T,)

    li = layer_input.reshape(N, D)
    ch = current_hidden_states.reshape(N, D)
    mkT = memory_keys.T
    wg1a = Wg1[:D]
    wg1b = Wg1[D:]
    wg2 = Wg2.reshape(1, H)
    bq2 = bq.reshape(1, D)
    bg1_2 = bg1.reshape(1, H)
    bg2_2 = bg2.reshape(1, 1)
    gamma = ln_gamma.reshape(1, D)
    beta = ln_beta.reshape(1, D)

    row_spec = pl.BlockSpec((T, D), lambda i: (i, 0))
    full = lambda shape: pl.BlockSpec(shape, lambda i: (0, 0))

    out, gate = pl.pallas_call(
        _fused_kernel,
        grid=grid,
        in_specs=[
            row_spec,
            row_spec,
            full((D, D)),
            full((1, D)),
            full((D, M)),
            full((M, D)),
            full((D, H)),
            full((D, H)),
            full((1, H)),
            full((1, H)),
            full((1, 1)),
            full((1, D)),
            full((1, D)),
        ],
        out_specs=[
            pl.BlockSpec((T, D), lambda i: (i, 0)),
            pl.BlockSpec((1, 1, T), lambda i: (i, 0, 0)),
        ],
        out_shape=[
            jax.ShapeDtypeStruct((N, D), jnp.float32),
            jax.ShapeDtypeStruct((N // T, 1, T), jnp.float32),
        ],
        compiler_params=pltpu.CompilerParams(
            dimension_semantics=("arbitrary",),
        ),
    )(li, ch, Wq, bq2, mkT, memory_values, wg1a, wg1b, bg1_2, wg2, bg2_2,
      gamma, beta)

    updated = out.reshape(B, S, D)
    reuse_gate = gate.reshape(B, S, 1)
    return (updated, reuse_gate)
